# Initial kernel scaffold; baseline (speedup 1.0000x reference)
import jax
import jax.numpy as jnp
from jax import lax
from jax.experimental import pallas as pl
from jax.experimental.pallas import tpu as pltpu
from jax.experimental.pallas import tpu_sc as plsc

NU = 100000
NI = 50000
NNZ = 1600000
D = 64
L = 16
NS = 16
NC = 2
NSL = D // L

W = 128
SUB = 8
CHUNK = W * SUB

NNZ_PAD = ((NNZ + NS * CHUNK - 1) // (NS * CHUNK)) * (NS * CHUNK)
NU_PAD = 100352
NI_PAD = 51200
RB = 64

_MESH = plsc.VectorSubcoreMesh(core_axis_name="c", subcore_axis_name="s")


def _worker_id():
    return lax.axis_index("s") * NC + lax.axis_index("c")


def _sqrt_newton(x):
    x = jnp.maximum(x, jnp.float32(1e-24))
    i = lax.bitcast_convert_type(x, jnp.int32)
    y = lax.bitcast_convert_type(jnp.int32(0x5F3759DF) - (i >> 1), jnp.float32)
    for _ in range(3):
        y = y * (jnp.float32(1.5) - jnp.float32(0.5) * x * y * y)
    return x * y


def _make_spmm(n_src_pad, n_dst_pad):
    rows_pt = n_dst_pad // NS
    e_pt = NNZ_PAD // NS
    n_chunks = e_pt // CHUNK
    sub_rows_pt = e_pt // W

    def body(x_hbm, src_hbm, dst_hbm, val_hbm, out_hbm,
             idx_s, idx_d, vals_v, gbuf, zbuf, acc, sem):
        cid = lax.axis_index("c")
        tid = lax.axis_index("s")

        def zb(j, _):
            zbuf[j, :] = jnp.zeros((L,), jnp.float32)
            return 0
        lax.fori_loop(0, CHUNK, zb, 0)

        row0 = tid * sub_rows_pt
        acc_off = tid * rows_pt

        for p in range(NSL // NC):
            sl = cid * (NSL // NC) + p

            done = 0
            while done < rows_pt:
                n = min(CHUNK, rows_pt - done)
                pltpu.sync_copy(zbuf.at[pl.ds(0, n)],
                                acc.at[pl.ds(acc_off + done, n)])
                done += n
            plsc.subcore_barrier()

            def chunk(k, _):
                r0 = row0 + k * SUB
                pltpu.sync_copy(src_hbm.at[pl.ds(r0, SUB)], idx_s)
                pltpu.sync_copy(dst_hbm.at[pl.ds(r0, SUB)], idx_d)
                pltpu.sync_copy(val_hbm.at[pl.ds(r0, SUB)], vals_v)
                cps = [pltpu.async_copy(x_hbm.at[sl].at[idx_s.at[b]],
                                        gbuf.at[pl.ds(b * W, W)], sem)
                       for b in range(SUB)]
                for c in cps:
                    c.wait()
                for b in range(SUB):
                    def mul(j, _, b=b):
                        r = b * W + j
                        gbuf[r, :] = gbuf[r, :] * vals_v[b, j]
                        return 0
                    lax.fori_loop(0, W, mul, 0)
                    pltpu.sync_copy(gbuf.at[pl.ds(b * W, W)],
                                    acc.at[idx_d.at[b]], add=True)
                return 0
            lax.fori_loop(0, n_chunks, chunk, 0)
            plsc.subcore_barrier()

            pltpu.sync_copy(acc.at[pl.ds(acc_off, rows_pt)],
                            out_hbm.at[sl].at[pl.ds(acc_off, rows_pt)])
            plsc.subcore_barrier()

    return pl.kernel(
        body,
        out_type=jax.ShapeDtypeStruct((NSL, n_dst_pad, L), jnp.float32),
        mesh=_MESH,
        scratch_types=[
            pltpu.VMEM((SUB, W), jnp.int32),
            pltpu.VMEM((SUB, W), jnp.int32),
            pltpu.VMEM((SUB, W), jnp.float32),
            pltpu.VMEM((CHUNK, L), jnp.float32),
            pltpu.VMEM((CHUNK, L), jnp.float32),
            pltpu.VMEM_SHARED((n_dst_pad, L), jnp.float32),
            pltpu.SemaphoreType.DMA,
        ],
    )


def _make_norm(n_pad):
    rpw = n_pad // (NS * NC)
    nblk = rpw // RB

    def body(x_hbm, out_hbm, xb):
        base = _worker_id() * rpw

        def blk(k, _):
            r0 = base + k * RB
            for c in range(NSL):
                pltpu.sync_copy(x_hbm.at[c].at[pl.ds(r0, RB)], xb.at[c])

            def row(j, _):
                v = [xb[c, j, :] for c in range(NSL)]
                ss_vec = v[0] * v[0] + v[1] * v[1] + v[2] * v[2] + v[3] * v[3]
                ss = jnp.sum(ss_vec)
                inv = jnp.float32(1.0) / jnp.maximum(_sqrt_newton(ss),
                                                     jnp.float32(1e-12))
                for c in range(NSL):
                    xb[c, j, :] = v[c] * inv
                return 0
            lax.fori_loop(0, RB, row, 0)

            for c in range(NSL):
                pltpu.sync_copy(xb.at[c], out_hbm.at[c].at[pl.ds(r0, RB)])
            return 0
        lax.fori_loop(0, nblk, blk, 0)

    return pl.kernel(
        body,
        out_type=jax.ShapeDtypeStruct((NSL, n_pad, L), jnp.float32),
        mesh=_MESH,
        scratch_types=[pltpu.VMEM((NSL, RB, L), jnp.float32)],
    )


def _make_slicer(n_pad):
    rpw = n_pad // (NS * NC)
    nblk = rpw // RB

    def body(x_hbm, out_hbm, xb):
        base = _worker_id() * rpw

        def blk(k, _):
            r0 = base + k * RB
            pltpu.sync_copy(x_hbm.at[pl.ds(r0, RB)], xb)
            for c in range(NSL):
                pltpu.sync_copy(xb.at[:, pl.ds(c * L, L)],
                                out_hbm.at[c].at[pl.ds(r0, RB)])
            return 0
        lax.fori_loop(0, nblk, blk, 0)

    return pl.kernel(
        body,
        out_type=jax.ShapeDtypeStruct((NSL, n_pad, L), jnp.float32),
        mesh=_MESH,
        scratch_types=[pltpu.VMEM((RB, D), jnp.float32)],
    )


def _final_body(ego_hbm, s1_hbm, s2_hbm, u3_hbm, out_hbm, eb, sb1, sb2, ub3):
    rpw = NU_PAD // (NS * NC)
    nblk = rpw // RB
    base = _worker_id() * rpw

    def blk(k, _):
        r0 = base + k * RB
        pltpu.sync_copy(ego_hbm.at[pl.ds(r0, RB)], eb)
        for c in range(NSL):
            pltpu.sync_copy(s1_hbm.at[c].at[pl.ds(r0, RB)], sb1.at[c])
            pltpu.sync_copy(s2_hbm.at[c].at[pl.ds(r0, RB)], sb2.at[c])
            pltpu.sync_copy(u3_hbm.at[c].at[pl.ds(r0, RB)], ub3.at[c])

        def row(j, _):
            u = [ub3[c, j, :] for c in range(NSL)]
            ss_vec = u[0] * u[0] + u[1] * u[1] + u[2] * u[2] + u[3] * u[3]
            ss = jnp.sum(ss_vec)
            inv = jnp.float32(1.0) / jnp.maximum(_sqrt_newton(ss),
                                                 jnp.float32(1e-12))
            for c in range(NSL):
                s3 = u[c] * inv
                acc = eb[j, pl.ds(c * L, L)] + sb1[c, j, :] + sb2[c, j, :] + s3
                eb[j, pl.ds(c * L, L)] = acc * jnp.float32(0.25)
            return 0
        lax.fori_loop(0, RB, row, 0)

        pltpu.sync_copy(eb, out_hbm.at[pl.ds(r0, RB)])
        return 0
    lax.fori_loop(0, nblk, blk, 0)


_spmm_A = _make_spmm(NI_PAD, NU_PAD)
_spmm_At = _make_spmm(NU_PAD, NI_PAD)
_norm_u = _make_norm(NU_PAD)
_norm_i = _make_norm(NI_PAD)
_slicer_u = _make_slicer(NU_PAD)
_slicer_i = _make_slicer(NI_PAD)
_final = pl.kernel(
    _final_body,
    out_type=jax.ShapeDtypeStruct((NU_PAD, D), jnp.float32),
    mesh=_MESH,
    scratch_types=[
        pltpu.VMEM((RB, D), jnp.float32),
        pltpu.VMEM((NSL, RB, L), jnp.float32),
        pltpu.VMEM((NSL, RB, L), jnp.float32),
        pltpu.VMEM((NSL, RB, L), jnp.float32),
    ],
)


@jax.jit
def kernel(ego_embeddings, nei_embeddings, A_rows, A_cols, A_vals):
    ego_p = jnp.pad(ego_embeddings, ((0, NU_PAD - NU), (0, 0)))
    nei_p = jnp.pad(nei_embeddings, ((0, NI_PAD - NI), (0, 0)))
    padn = NNZ_PAD - NNZ
    ar = jnp.concatenate([A_rows, jnp.arange(padn, dtype=jnp.int32) % NU])
    ac = jnp.concatenate([A_cols, jnp.arange(padn, dtype=jnp.int32) % NI])
    av = jnp.concatenate([A_vals, jnp.zeros((padn,), jnp.float32)])
    rows2 = ar.reshape(-1, W)
    cols2 = ac.reshape(-1, W)
    vals2 = av.reshape(-1, W)

    ego_sl = _slicer_u(ego_p)
    nei_sl = _slicer_i(nei_p)

    u1 = _spmm_A(nei_sl, cols2, rows2, vals2)
    s1 = _norm_u(u1)
    v1 = _spmm_At(ego_sl, rows2, cols2, vals2)
    t1 = _norm_i(v1)
    u2 = _spmm_A(t1, cols2, rows2, vals2)
    s2 = _norm_u(u2)
    v2 = _spmm_At(s1, rows2, cols2, vals2)
    t2 = _norm_i(v2)
    u3 = _spmm_A(t2, cols2, rows2, vals2)
    out = _final(ego_p, s1, s2, u3)
    return out[:NU]


# speedup vs baseline: 4.9119x; 4.9119x over previous
import jax
import jax.numpy as jnp
from jax import lax
from jax.experimental import pallas as pl
from jax.experimental.pallas import tpu as pltpu
from jax.experimental.pallas import tpu_sc as plsc

NU = 100000
NI = 50000
NNZ = 1600000
D = 64
L = 16
NS = 16
NC = 2
NSL = D // L

W = 128
SUB = 8
CHUNK = W * SUB

NNZ_PAD = ((NNZ + NS * CHUNK - 1) // (NS * CHUNK)) * (NS * CHUNK)
NU_PAD = 100352
NI_PAD = 51200
RB = 64

_MESH = plsc.VectorSubcoreMesh(core_axis_name="c", subcore_axis_name="s")


def _worker_id():
    return lax.axis_index("s") * NC + lax.axis_index("c")


def _rsqrt_newton(x):
    x = jnp.maximum(x, jnp.float32(1e-24))
    i = lax.bitcast_convert_type(x, jnp.int32)
    y = lax.bitcast_convert_type(jnp.int32(0x5F3759DF) - (i >> 1), jnp.float32)
    for _ in range(3):
        y = y * (jnp.float32(1.5) - jnp.float32(0.5) * x * y * y)
    return y


def _make_spmm(n_src_pad, n_dst_pad):
    rows_pt = n_dst_pad // NS
    e_pt = NNZ_PAD // NS
    n_chunks = e_pt // CHUNK
    sub_rows_pt = e_pt // W

    def body(x_hbm, src_hbm, dst_hbm, val_hbm, out_hbm,
             idx_s, idx_d, vals_v, gbuf, acc, sem):
        cid = lax.axis_index("c")
        tid = lax.axis_index("s")

        row0 = tid * sub_rows_pt
        acc_off = tid * rows_pt

        for p in range(NSL // NC):
            sl = cid * (NSL // NC) + p

            def zb(j, _):
                gbuf[j, :] = jnp.zeros((L,), jnp.float32)
                return 0
            lax.fori_loop(0, CHUNK, zb, 0)

            done = 0
            while done < rows_pt:
                n = min(CHUNK, rows_pt - done)
                pltpu.sync_copy(gbuf.at[pl.ds(0, n)],
                                acc.at[pl.ds(acc_off + done, n)])
                done += n
            plsc.subcore_barrier()

            def chunk(k, _):
                r0 = row0 + k * SUB
                pltpu.sync_copy(src_hbm.at[pl.ds(r0, SUB)], idx_s)
                pltpu.sync_copy(dst_hbm.at[pl.ds(r0, SUB)], idx_d)
                pltpu.sync_copy(val_hbm.at[pl.ds(r0, SUB)], vals_v)
                cps = [pltpu.async_copy(x_hbm.at[sl].at[idx_s.at[b]],
                                        gbuf.at[pl.ds(b * W, W)], sem)
                       for b in range(SUB)]
                for c in cps:
                    c.wait()
                for b in range(SUB):
                    def mul(j, _, b=b):
                        vv = vals_v[b, pl.ds(j * L, L)]
                        for ll in range(L):
                            r = b * W + j * L + ll
                            gbuf[r, :] = gbuf[r, :] * vv[ll]
                        return 0
                    lax.fori_loop(0, W // L, mul, 0)
                    pltpu.sync_copy(gbuf.at[pl.ds(b * W, W)],
                                    acc.at[idx_d.at[b]], add=True)
                return 0
            lax.fori_loop(0, n_chunks, chunk, 0)
            plsc.subcore_barrier()

            pltpu.sync_copy(acc.at[pl.ds(acc_off, rows_pt)],
                            out_hbm.at[sl].at[pl.ds(acc_off, rows_pt)])
            plsc.subcore_barrier()

    return pl.kernel(
        body,
        out_type=jax.ShapeDtypeStruct((NSL, n_dst_pad, L), jnp.float32),
        mesh=_MESH,
        compiler_params=pltpu.CompilerParams(use_tc_tiling_on_sc=False, needs_layout_passes=False),
        scratch_types=[
            pltpu.VMEM((SUB, W), jnp.int32),
            pltpu.VMEM((SUB, W), jnp.int32),
            pltpu.VMEM((SUB, W), jnp.float32),
            pltpu.VMEM((CHUNK, L), jnp.float32),
            pltpu.VMEM_SHARED((n_dst_pad, L), jnp.float32),
            pltpu.SemaphoreType.DMA,
        ],
    )


def _make_norm(n_pad):
    rpw = n_pad // (NS * NC)
    nblk = rpw // RB

    def body(x_hbm, out_hbm, xb):
        base = _worker_id() * rpw

        def blk(k, _):
            r0 = base + k * RB
            for c in range(NSL):
                pltpu.sync_copy(x_hbm.at[c].at[pl.ds(r0, RB)], xb.at[c])

            def row(j, _):
                v = [xb[c, j, :] for c in range(NSL)]
                ss_vec = v[0] * v[0] + v[1] * v[1] + v[2] * v[2] + v[3] * v[3]
                ss = jnp.sum(ss_vec)
                inv = _rsqrt_newton(ss)
                for c in range(NSL):
                    xb[c, j, :] = v[c] * inv
                return 0
            lax.fori_loop(0, RB, row, 0)

            for c in range(NSL):
                pltpu.sync_copy(xb.at[c], out_hbm.at[c].at[pl.ds(r0, RB)])
            return 0
        lax.fori_loop(0, nblk, blk, 0)

    return pl.kernel(
        body,
        out_type=jax.ShapeDtypeStruct((NSL, n_pad, L), jnp.float32),
        mesh=_MESH,
        compiler_params=pltpu.CompilerParams(use_tc_tiling_on_sc=False, needs_layout_passes=False),
        scratch_types=[pltpu.VMEM((NSL, RB, L), jnp.float32)],
    )


def _make_slicer(n_pad):
    rpw = n_pad // (NS * NC)
    nblk = rpw // RB

    def body(x_hbm, out_hbm, xb):
        base = _worker_id() * rpw

        def blk(k, _):
            r0 = base + k * RB
            pltpu.sync_copy(x_hbm.at[pl.ds(r0, RB)], xb)
            for c in range(NSL):
                pltpu.sync_copy(xb.at[:, pl.ds(c * L, L)],
                                out_hbm.at[c].at[pl.ds(r0, RB)])
            return 0
        lax.fori_loop(0, nblk, blk, 0)

    return pl.kernel(
        body,
        out_type=jax.ShapeDtypeStruct((NSL, n_pad, L), jnp.float32),
        mesh=_MESH,
        compiler_params=pltpu.CompilerParams(use_tc_tiling_on_sc=False, needs_layout_passes=False),
        scratch_types=[pltpu.VMEM((RB, D), jnp.float32)],
    )


def _final_body(ego_hbm, s1_hbm, s2_hbm, u3_hbm, out_hbm, eb, sb1, sb2, ub3):
    rpw = NU_PAD // (NS * NC)
    nblk = rpw // RB
    base = _worker_id() * rpw

    def blk(k, _):
        r0 = base + k * RB
        pltpu.sync_copy(ego_hbm.at[pl.ds(r0, RB)], eb)
        for c in range(NSL):
            pltpu.sync_copy(s1_hbm.at[c].at[pl.ds(r0, RB)], sb1.at[c])
            pltpu.sync_copy(s2_hbm.at[c].at[pl.ds(r0, RB)], sb2.at[c])
            pltpu.sync_copy(u3_hbm.at[c].at[pl.ds(r0, RB)], ub3.at[c])

        def row(j, _):
            u = [ub3[c, j, :] for c in range(NSL)]
            ss_vec = u[0] * u[0] + u[1] * u[1] + u[2] * u[2] + u[3] * u[3]
            ss = jnp.sum(ss_vec)
            inv = _rsqrt_newton(ss)
            for c in range(NSL):
                s3 = u[c] * inv
                acc = eb[j, pl.ds(c * L, L)] + sb1[c, j, :] + sb2[c, j, :] + s3
                eb[j, pl.ds(c * L, L)] = acc * jnp.float32(0.25)
            return 0
        lax.fori_loop(0, RB, row, 0)

        pltpu.sync_copy(eb, out_hbm.at[pl.ds(r0, RB)])
        return 0
    lax.fori_loop(0, nblk, blk, 0)


_spmm_A = _make_spmm(NI_PAD, NU_PAD)
_spmm_At = _make_spmm(NU_PAD, NI_PAD)
_norm_u = _make_norm(NU_PAD)
_norm_i = _make_norm(NI_PAD)
_slicer_u = _make_slicer(NU_PAD)
_slicer_i = _make_slicer(NI_PAD)
_final = pl.kernel(
    _final_body,
    out_type=jax.ShapeDtypeStruct((NU_PAD, D), jnp.float32),
    mesh=_MESH,
    compiler_params=pltpu.CompilerParams(use_tc_tiling_on_sc=False, needs_layout_passes=False),
    scratch_types=[
        pltpu.VMEM((RB, D), jnp.float32),
        pltpu.VMEM((NSL, RB, L), jnp.float32),
        pltpu.VMEM((NSL, RB, L), jnp.float32),
        pltpu.VMEM((NSL, RB, L), jnp.float32),
    ],
)


@jax.jit
def kernel(ego_embeddings, nei_embeddings, A_rows, A_cols, A_vals):
    ego_p = jnp.pad(ego_embeddings, ((0, NU_PAD - NU), (0, 0)))
    nei_p = jnp.pad(nei_embeddings, ((0, NI_PAD - NI), (0, 0)))
    padn = NNZ_PAD - NNZ
    ar = jnp.concatenate([A_rows, jnp.arange(padn, dtype=jnp.int32) % NU])
    ac = jnp.concatenate([A_cols, jnp.arange(padn, dtype=jnp.int32) % NI])
    av = jnp.concatenate([A_vals, jnp.zeros((padn,), jnp.float32)])
    rows2 = ar.reshape(-1, W)
    cols2 = ac.reshape(-1, W)
    vals2 = av.reshape(-1, W)

    ego_sl = _slicer_u(ego_p)
    nei_sl = _slicer_i(nei_p)

    u1 = _spmm_A(nei_sl, cols2, rows2, vals2)
    s1 = _norm_u(u1)
    v1 = _spmm_At(ego_sl, rows2, cols2, vals2)
    t1 = _norm_i(v1)
    u2 = _spmm_A(t1, cols2, rows2, vals2)
    s2 = _norm_u(u2)
    v2 = _spmm_At(s1, rows2, cols2, vals2)
    t2 = _norm_i(v2)
    u3 = _spmm_A(t2, cols2, rows2, vals2)
    out = _final(ego_p, s1, s2, u3)
    return out[:NU]
